# baseline (speedup 1.0000x reference)
import functools

import jax
import jax.numpy as jnp
from jax.experimental import pallas as pl
from jax.experimental.pallas import tpu as pltpu

EPS = 1e-4
_HB_STATS = 128
_HB = 64


def _stats_kernel(x_ref, o_ref):
    h = pl.program_id(1)
    D = o_ref.shape[2]
    comps = [x_ref[0, q * D:(q + 1) * D] for q in range(4)]
    vals = [jnp.sum(c, axis=1) for c in comps]
    for a in range(4):
        for b in range(a, 4):
            vals.append(jnp.sum(comps[a] * comps[b], axis=1))

    @pl.when(h == 0)
    def _():
        for idx, v in enumerate(vals):
            o_ref[0, idx] = v

    @pl.when(h != 0)
    def _():
        for idx, v in enumerate(vals):
            o_ref[0, idx] += v


def _prep_kernel(part_ref, g_ref, b_ref, o_ref, *, n_total):
    t = jnp.sum(part_ref[...], axis=0)
    T = jnp.sum(t, axis=2)
    inv_n = 1.0 / n_total

    def row(i):
        return T[i:i + 1, :] * inv_n

    mu = [row(q) for q in range(4)]
    pidx = {}
    cnt = 4
    for a in range(4):
        for b in range(a, 4):
            pidx[(a, b)] = cnt
            cnt += 1

    def V(a, b):
        v = row(pidx[(a, b)]) - mu[a] * mu[b]
        return v + EPS if a == b else v

    Vrr, Vri, Vrj, Vrk = V(0, 0), V(0, 1), V(0, 2), V(0, 3)
    Vii, Vij, Vik = V(1, 1), V(1, 2), V(1, 3)
    Vjj, Vjk, Vkk = V(2, 2), V(2, 3), V(3, 3)

    Wrr = jnp.sqrt(Vrr)
    Wri = Vri / Wrr
    Wii = jnp.sqrt(Vii - Wri * Wri)
    Wrj = Vrj / Wrr
    Wij = (Vij - Wri * Wrj) / Wii
    Wjj = jnp.sqrt(Vjj - (Wij * Wij + Wrj * Wrj))
    Wrk = Vrk / Wrr
    Wik = (Vik - Wri * Wrk) / Wii
    Wjk = (Vjk - (Wij * Wik + Wrj * Wrk)) / Wjj
    Wkk = jnp.sqrt(Vkk - (Wjk * Wjk + Wik * Wik + Wrk * Wrk))

    Wm = [[Wrr, Wri, Wrj, Wrk],
          [Wri, Wii, Wij, Wik],
          [Wrj, Wij, Wjj, Wjk],
          [Wrk, Wik, Wjk, Wkk]]

    def g(i):
        return g_ref[i:i + 1, :]
    Gm = [[g(0), g(1), g(2), g(3)],
          [g(1), g(4), g(5), g(6)],
          [g(2), g(5), g(7), g(8)],
          [g(3), g(6), g(8), g(9)]]

    for q in range(4):
        for p in range(4):
            m = Gm[q][0] * Wm[0][p]
            for s in range(1, 4):
                m = m + Gm[q][s] * Wm[s][p]
            o_ref[4 * q + p:4 * q + p + 1, :] = m
    for q in range(4):
        c = b_ref[q:q + 1, :]
        for p in range(4):
            c = c - o_ref[4 * q + p:4 * q + p + 1, :] * mu[p]
        o_ref[16 + q:17 + q, :] = c


def _apply_kernel(p_ref, x_ref, o_ref):
    D = p_ref.shape[1]
    for d in range(D):
        xs = [x_ref[0, p * D + d] for p in range(4)]
        for q in range(4):
            acc = xs[0] * p_ref[4 * q, d]
            for p in range(1, 4):
                acc = acc + xs[p] * p_ref[4 * q + p, d]
            o_ref[0, q * D + d] = acc + p_ref[16 + q, d]


def kernel(x, gamma_rr, gamma_ri, gamma_rj, gamma_rk, gamma_ii, gamma_ij,
           gamma_ik, gamma_jj, gamma_jk, gamma_kk, beta):
    B, C, H, W = x.shape
    D = C // 4
    n_total = B * H * W
    hsteps = H // _HB

    partial = pl.pallas_call(
        _stats_kernel,
        grid=(B, H // _HB_STATS),
        in_specs=[pl.BlockSpec((1, C, _HB_STATS, W), lambda b, h: (b, 0, h, 0))],
        out_specs=pl.BlockSpec((1, 14, D, W), lambda b, h: (b, 0, 0, 0)),
        out_shape=jax.ShapeDtypeStruct((B, 14, D, W), jnp.float32),
        compiler_params=pltpu.CompilerParams(
            dimension_semantics=("parallel", "arbitrary"),
            vmem_limit_bytes=50 * 1024 * 1024,
        ),
        name="qbn_stats",
    )(x)

    gmat = jnp.stack([gamma_rr, gamma_ri, gamma_rj, gamma_rk, gamma_ii,
                      gamma_ij, gamma_ik, gamma_jj, gamma_jk, gamma_kk],
                     axis=0)
    bmat = beta.reshape(4, D)

    params = pl.pallas_call(
        functools.partial(_prep_kernel, n_total=float(n_total)),
        out_shape=jax.ShapeDtypeStruct((20, D), jnp.float32),
        name="qbn_prep",
    )(partial, gmat, bmat)

    out = pl.pallas_call(
        _apply_kernel,
        grid=(B, hsteps),
        in_specs=[
            pl.BlockSpec(memory_space=pltpu.SMEM),
            pl.BlockSpec((1, C, _HB, W), lambda b, h: (b, 0, h, 0)),
        ],
        out_specs=pl.BlockSpec((1, C, _HB, W), lambda b, h: (b, 0, h, 0)),
        out_shape=jax.ShapeDtypeStruct((B, C, H, W), jnp.float32),
        compiler_params=pltpu.CompilerParams(
            dimension_semantics=("parallel", "arbitrary"),
            vmem_limit_bytes=50 * 1024 * 1024,
        ),
        name="qbn_apply",
    )(params, x)
    return out


# speedup vs baseline: 1.0312x; 1.0312x over previous
import functools

import jax
import jax.numpy as jnp
from jax.experimental import pallas as pl
from jax.experimental.pallas import tpu as pltpu

EPS = 1e-4
_HB_STATS = 128
_HB = 128


def _stats_kernel(x_ref, o_ref):
    h = pl.program_id(1)
    D = o_ref.shape[2]
    comps = [x_ref[0, q * D:(q + 1) * D] for q in range(4)]
    vals = [jnp.sum(c, axis=1) for c in comps]
    for a in range(4):
        for b in range(a, 4):
            vals.append(jnp.sum(comps[a] * comps[b], axis=1))

    @pl.when(h == 0)
    def _():
        for idx, v in enumerate(vals):
            o_ref[0, idx] = v

    @pl.when(h != 0)
    def _():
        for idx, v in enumerate(vals):
            o_ref[0, idx] += v


def _prep_kernel(part_ref, *refs, n_total):
    g_refs = refs[:10]
    beta_ref = refs[10]
    o_ref = refs[11]
    D = o_ref.shape[1]
    t = jnp.sum(part_ref[...], axis=0)
    T = jnp.sum(t, axis=2)
    inv_n = 1.0 / n_total

    def row(i):
        return T[i:i + 1, :] * inv_n

    mu = [row(q) for q in range(4)]
    pidx = {}
    cnt = 4
    for a in range(4):
        for b in range(a, 4):
            pidx[(a, b)] = cnt
            cnt += 1

    def V(a, b):
        v = row(pidx[(a, b)]) - mu[a] * mu[b]
        return v + EPS if a == b else v

    Vrr, Vri, Vrj, Vrk = V(0, 0), V(0, 1), V(0, 2), V(0, 3)
    Vii, Vij, Vik = V(1, 1), V(1, 2), V(1, 3)
    Vjj, Vjk, Vkk = V(2, 2), V(2, 3), V(3, 3)

    Wrr = jnp.sqrt(Vrr)
    Wri = Vri / Wrr
    Wii = jnp.sqrt(Vii - Wri * Wri)
    Wrj = Vrj / Wrr
    Wij = (Vij - Wri * Wrj) / Wii
    Wjj = jnp.sqrt(Vjj - (Wij * Wij + Wrj * Wrj))
    Wrk = Vrk / Wrr
    Wik = (Vik - Wri * Wrk) / Wii
    Wjk = (Vjk - (Wij * Wik + Wrj * Wrk)) / Wjj
    Wkk = jnp.sqrt(Vkk - (Wjk * Wjk + Wik * Wik + Wrk * Wrk))

    Wm = [[Wrr, Wri, Wrj, Wrk],
          [Wri, Wii, Wij, Wik],
          [Wrj, Wij, Wjj, Wjk],
          [Wrk, Wik, Wjk, Wkk]]

    def g(i):
        return g_refs[i][...].reshape(1, D)
    Gm = [[g(0), g(1), g(2), g(3)],
          [g(1), g(4), g(5), g(6)],
          [g(2), g(5), g(7), g(8)],
          [g(3), g(6), g(8), g(9)]]

    for q in range(4):
        for p in range(4):
            m = Gm[q][0] * Wm[0][p]
            for s in range(1, 4):
                m = m + Gm[q][s] * Wm[s][p]
            o_ref[4 * q + p:4 * q + p + 1, :] = m
    for q in range(4):
        c = beta_ref[q * D:(q + 1) * D].reshape(1, D)
        for p in range(4):
            c = c - o_ref[4 * q + p:4 * q + p + 1, :] * mu[p]
        o_ref[16 + q:17 + q, :] = c


def _apply_kernel(p_ref, x_ref, o_ref):
    D = p_ref.shape[1]
    for d in range(D):
        xs = [x_ref[0, p * D + d] for p in range(4)]
        for q in range(4):
            acc = xs[0] * p_ref[4 * q, d]
            for p in range(1, 4):
                acc = acc + xs[p] * p_ref[4 * q + p, d]
            o_ref[0, q * D + d] = acc + p_ref[16 + q, d]


def kernel(x, gamma_rr, gamma_ri, gamma_rj, gamma_rk, gamma_ii, gamma_ij,
           gamma_ik, gamma_jj, gamma_jk, gamma_kk, beta):
    B, C, H, W = x.shape
    D = C // 4
    n_total = B * H * W
    hsteps = H // _HB

    partial = pl.pallas_call(
        _stats_kernel,
        grid=(B, H // _HB_STATS),
        in_specs=[pl.BlockSpec((1, C, _HB_STATS, W), lambda b, h: (b, 0, h, 0))],
        out_specs=pl.BlockSpec((1, 14, D, W), lambda b, h: (b, 0, 0, 0)),
        out_shape=jax.ShapeDtypeStruct((B, 14, D, W), jnp.float32),
        compiler_params=pltpu.CompilerParams(
            dimension_semantics=("parallel", "arbitrary"),
            vmem_limit_bytes=50 * 1024 * 1024,
        ),
        name="qbn_stats",
    )(x)

    params = pl.pallas_call(
        functools.partial(_prep_kernel, n_total=float(n_total)),
        out_shape=jax.ShapeDtypeStruct((20, D), jnp.float32),
        name="qbn_prep",
    )(partial, gamma_rr, gamma_ri, gamma_rj, gamma_rk, gamma_ii, gamma_ij,
      gamma_ik, gamma_jj, gamma_jk, gamma_kk, beta)

    out = pl.pallas_call(
        _apply_kernel,
        grid=(B, hsteps),
        in_specs=[
            pl.BlockSpec(memory_space=pltpu.SMEM),
            pl.BlockSpec((1, C, _HB, W), lambda b, h: (b, 0, h, 0)),
        ],
        out_specs=pl.BlockSpec((1, C, _HB, W), lambda b, h: (b, 0, h, 0)),
        out_shape=jax.ShapeDtypeStruct((B, C, H, W), jnp.float32),
        compiler_params=pltpu.CompilerParams(
            dimension_semantics=("parallel", "arbitrary"),
            vmem_limit_bytes=50 * 1024 * 1024,
        ),
        name="qbn_apply",
    )(params, x)
    return out
